# Initial kernel scaffold; baseline (speedup 1.0000x reference)
import jax
import jax.numpy as jnp
from jax.experimental import pallas as pl


def kernel(x, table, W1, b1, W2, b2, W3, b3):
    raise NotImplementedError("write your pallas kernel here")


# speedup vs baseline: 1.9504x; 1.9504x over previous
import functools

import jax
import jax.numpy as jnp
from jax import lax
from jax.experimental import pallas as pl
from jax.experimental.pallas import tpu as pltpu
from jax.experimental.pallas import tpu_sc as plsc

B = 16384
L = 200
EMB = 64
OUT = 128

NC = 2
NS = 16
NW = NC * NS
RW = B // NW
CB = 8
NCHUNK = RW // CB
HALF = L // 2

_sc_mesh = plsc.VectorSubcoreMesh(core_axis_name="c", subcore_axis_name="s")


@functools.partial(
    pl.kernel,
    mesh=_sc_mesh,
    out_type=jax.ShapeDtypeStruct((B, EMB), jnp.float32),
    scratch_types=[
        pltpu.VMEM((2 * CB, HALF), jnp.int32),
        pltpu.VMEM((L, EMB), jnp.float32),
        pltpu.VMEM((CB, EMB), jnp.float32),
        pltpu.SemaphoreType.DMA,
    ],
    compiler_params=pltpu.CompilerParams(use_tc_tiling_on_sc=False),
)
def _sc_pool(idx_hbm, table_hbm, out_hbm, idxv, buf, accv, sem):
    wid = lax.axis_index("s") * NC + lax.axis_index("c")
    inv_l = jnp.float32(1.0 / L)

    def chunk_body(c, _):
        pltpu.sync_copy(idx_hbm.at[wid, pl.ds(c * (2 * CB), 2 * CB)], idxv)

        def row_body(r, _):
            cp0 = pltpu.async_copy(
                table_hbm.at[idxv.at[2 * r]], buf.at[pl.ds(0, HALF)], sem)
            cp1 = pltpu.async_copy(
                table_hbm.at[idxv.at[2 * r + 1]], buf.at[pl.ds(HALF, HALF)], sem)
            cp0.wait()
            cp1.wait()

            def acc_body(j, accs):
                return tuple(
                    accs[k] + buf[j, pl.ds(16 * k, 16)] for k in range(4))

            zero = jnp.zeros((16,), jnp.float32)
            acc = lax.fori_loop(0, L, acc_body, (zero, zero, zero, zero))
            for k in range(4):
                accv[r, pl.ds(16 * k, 16)] = acc[k] * inv_l
            return 0

        lax.fori_loop(0, CB, row_body, 0)
        pltpu.sync_copy(accv, out_hbm.at[pl.ds(wid * RW + c * CB, CB)])
        return 0

    lax.fori_loop(0, NCHUNK, chunk_body, 0)


def _mlp_body(p_ref, w1_ref, b1_ref, w2_ref, b2_ref, w3_ref, b3_ref, o_ref):
    h = jnp.dot(p_ref[...], w1_ref[...], preferred_element_type=jnp.float32)
    h = jnp.maximum(h + b1_ref[...], 0.0)
    h = jnp.dot(h, w2_ref[...], preferred_element_type=jnp.float32)
    h = jnp.maximum(h + b2_ref[...], 0.0)
    o_ref[...] = (
        jnp.dot(h, w3_ref[...], preferred_element_type=jnp.float32)
        + b3_ref[...])


_BT = 1024


def _mlp(pooled, W1, b1, W2, b2, W3, b3):
    full = lambda shape: pl.BlockSpec(shape, lambda i: (0, 0))
    return pl.pallas_call(
        _mlp_body,
        grid=(B // _BT,),
        in_specs=[
            pl.BlockSpec((_BT, EMB), lambda i: (i, 0)),
            full(W1.shape), full((1, 512)),
            full(W2.shape), full((1, 256)),
            full(W3.shape), full((1, OUT)),
        ],
        out_specs=pl.BlockSpec((_BT, OUT), lambda i: (i, 0)),
        out_shape=jax.ShapeDtypeStruct((B, OUT), jnp.float32),
    )(pooled, W1, b1.reshape(1, 512), W2, b2.reshape(1, 256),
      W3, b3.reshape(1, OUT))


def kernel(x, table, W1, b1, W2, b2, W3, b3):
    idx = x.astype(jnp.int32).reshape(NW, 2 * RW, HALF)
    pooled = _sc_pool(idx, table)
    return _mlp(pooled, W1, b1, W2, b2, W3, b3)


# speedup vs baseline: 2.8650x; 1.4690x over previous
import functools

import jax
import jax.numpy as jnp
from jax import lax
from jax.experimental import pallas as pl
from jax.experimental.pallas import tpu as pltpu
from jax.experimental.pallas import tpu_sc as plsc

B = 16384
L = 200
EMB = 64
OUT = 128

NC = 2
NS = 16
NW = NC * NS
NL = 2
BH = B // NL
RW = BH // NW
HALF = L // 2
ACC = 8

_sc_mesh = plsc.VectorSubcoreMesh(core_axis_name="c", subcore_axis_name="s")


def _accumulate(buf_ref):
    zero = jnp.zeros((16,), jnp.float32)

    def body(j, accs):
        res = list(accs)
        for jj in range(8):
            row = j * 8 + jj
            for k in range(4):
                res[(jj % 2) * 4 + k] = (
                    res[(jj % 2) * 4 + k] + buf_ref[row, pl.ds(16 * k, 16)])
        return tuple(res)

    accs = lax.fori_loop(0, L // 8, body, (zero,) * 8)
    return [accs[k] + accs[k + 4] for k in range(4)]


@functools.partial(
    pl.kernel,
    mesh=_sc_mesh,
    out_type=jax.ShapeDtypeStruct((BH, EMB), jnp.float32),
    scratch_types=[
        pltpu.VMEM((2 * RW, HALF), jnp.int32),
        pltpu.VMEM((L, EMB), jnp.float32),
        pltpu.VMEM((L, EMB), jnp.float32),
        pltpu.VMEM((ACC, EMB), jnp.float32),
        pltpu.SemaphoreType.DMA,
        pltpu.SemaphoreType.DMA,
        pltpu.SemaphoreType.DMA,
    ],
    compiler_params=pltpu.CompilerParams(use_tc_tiling_on_sc=False),
)
def _sc_pool(idx_hbm, table_hbm, out_hbm, idxv, bufa, bufb, accv,
             sema, semb, semo):
    wid = lax.axis_index("s") * NC + lax.axis_index("c")
    inv_l = jnp.float32(1.0 / L)
    half_acc = ACC // 2

    pltpu.sync_copy(idx_hbm.at[wid], idxv)

    def fire(row, buf, sem):
        pltpu.async_copy(table_hbm.at[idxv.at[2 * row]],
                         buf.at[pl.ds(0, HALF)], sem)
        pltpu.async_copy(table_hbm.at[idxv.at[2 * row + 1]],
                         buf.at[pl.ds(HALF, HALF)], sem)

    def drain(buf, sem):
        pltpu.make_async_copy(table_hbm.at[pl.ds(0, L)], buf, sem).wait()

    def finish(row, buf):
        acc = _accumulate(buf)
        for k in range(4):
            accv[row % ACC, pl.ds(16 * k, 16)] = acc[k] * inv_l

    fire(0, bufa, sema)

    def body(u, _):
        r0 = 2 * u
        fire(r0 + 1, bufb, semb)
        drain(bufa, sema)
        finish(r0, bufa)

        @pl.when(u < RW // 2 - 1)
        def _():
            fire(r0 + 2, bufa, sema)

        drain(bufb, semb)
        finish(r0 + 1, bufb)

        @pl.when(u % 2 == 1)
        def _():
            side = (u // 2) % 2

            @pl.when(u >= 5)
            def _():
                pltpu.make_async_copy(
                    accv.at[pl.ds(0, half_acc)],
                    out_hbm.at[pl.ds(wid * RW, half_acc)], semo).wait()

            pltpu.async_copy(
                accv.at[pl.ds(side * half_acc, half_acc)],
                out_hbm.at[pl.ds(wid * RW + r0 - 2, half_acc)], semo)

        return 0

    lax.fori_loop(0, RW // 2, body, 0)
    for _ in range(2):
        pltpu.make_async_copy(
            accv.at[pl.ds(0, half_acc)],
            out_hbm.at[pl.ds(wid * RW, half_acc)], semo).wait()


def _mlp_body(p_ref, w1_ref, b1_ref, w2_ref, b2_ref, w3_ref, b3_ref, o_ref):
    h = jnp.dot(p_ref[...], w1_ref[...], preferred_element_type=jnp.float32)
    h = jnp.maximum(h + b1_ref[...], 0.0)
    h = jnp.dot(h, w2_ref[...], preferred_element_type=jnp.float32)
    h = jnp.maximum(h + b2_ref[...], 0.0)
    o_ref[...] = (
        jnp.dot(h, w3_ref[...], preferred_element_type=jnp.float32)
        + b3_ref[...])


_BT = 1024


def _mlp(pooled, W1, b1, W2, b2, W3, b3):
    full = lambda shape: pl.BlockSpec(shape, lambda i: (0, 0))
    return pl.pallas_call(
        _mlp_body,
        grid=(B // _BT,),
        in_specs=[
            pl.BlockSpec((_BT, EMB), lambda i: (i, 0)),
            full(W1.shape), full((1, 512)),
            full(W2.shape), full((1, 256)),
            full(W3.shape), full((1, OUT)),
        ],
        out_specs=pl.BlockSpec((_BT, OUT), lambda i: (i, 0)),
        out_shape=jax.ShapeDtypeStruct((B, OUT), jnp.float32),
    )(pooled, W1, b1.reshape(1, 512), W2, b2.reshape(1, 256),
      W3, b3.reshape(1, OUT))


def kernel(x, table, W1, b1, W2, b2, W3, b3):
    idx = x.astype(jnp.int32).reshape(NL, NW, 2 * RW, HALF)
    pooled = jnp.concatenate(
        [_sc_pool(idx[h], table) for h in range(NL)], axis=0)
    return _mlp(pooled, W1, b1, W2, b2, W3, b3)


# speedup vs baseline: 3.4057x; 1.1887x over previous
import functools

import jax
import jax.numpy as jnp
from jax import lax
from jax.experimental import pallas as pl
from jax.experimental.pallas import tpu as pltpu
from jax.experimental.pallas import tpu_sc as plsc

B = 16384
L = 200
EMB = 64
OUT = 128

NC = 2
NS = 16
NW = NC * NS
RW = B // NW
HALF = L // 2
ACC = 8
QI = 256
NQ = (2 * RW) // QI
VB = RW // 4

_sc_mesh = plsc.VectorSubcoreMesh(core_axis_name="c", subcore_axis_name="s")


def _accumulate(buf_ref):
    zero = jnp.zeros((16,), jnp.float32)

    def body(j, accs):
        res = list(accs)
        for jj in range(8):
            row = j * 8 + jj
            for k in range(4):
                res[(jj % 2) * 4 + k] = (
                    res[(jj % 2) * 4 + k] + buf_ref[row, pl.ds(16 * k, 16)])
        return tuple(res)

    accs = lax.fori_loop(0, L // 8, body, (zero,) * 8)
    return [accs[k] + accs[k + 4] for k in range(4)]


@functools.partial(
    pl.kernel,
    mesh=_sc_mesh,
    out_type=jax.ShapeDtypeStruct((B, EMB), jnp.float32),
    scratch_types=[
        pltpu.VMEM((2, QI, HALF), jnp.int32),
        pltpu.VMEM((L, EMB), jnp.float32),
        pltpu.VMEM((L, EMB), jnp.float32),
        pltpu.VMEM((L, EMB), jnp.float32),
        pltpu.VMEM((L, EMB), jnp.float32),
        pltpu.VMEM((ACC, EMB), jnp.float32),
        pltpu.SemaphoreType.DMA,
        pltpu.SemaphoreType.DMA,
        pltpu.SemaphoreType.DMA,
        pltpu.SemaphoreType.DMA,
        pltpu.SemaphoreType.DMA,
        pltpu.SemaphoreType.DMA,
    ],
    compiler_params=pltpu.CompilerParams(use_tc_tiling_on_sc=False),
)
def _sc_pool(idx_hbm, table_hbm, out_hbm, idxq, buf0, buf1, buf2, buf3, accv,
             sem0, sem1, sem2, sem3, semi, semo):
    wid = lax.axis_index("s") * NC + lax.axis_index("c")
    inv_l = jnp.float32(1.0 / L)
    bufs = (buf0, buf1, buf2, buf3)
    sems = (sem0, sem1, sem2, sem3)

    def fire(row, slot):
        for h in range(2):
            i = 2 * row + h
            pltpu.async_copy(
                table_hbm.at[idxq.at[(i // QI) % 2, i % QI]],
                bufs[slot].at[pl.ds(h * HALF, HALF)], sems[slot])

    def drain(slot):
        pltpu.make_async_copy(
            table_hbm.at[pl.ds(0, L)], bufs[slot], sems[slot]).wait()

    def finish(row, slot):
        acc = _accumulate(bufs[slot])
        for k in range(4):
            accv[row % ACC, pl.ds(16 * k, 16)] = acc[k] * inv_l

    def drain_out():
        pltpu.make_async_copy(
            accv.at[pl.ds(0, ACC // 2)],
            out_hbm.at[pl.ds(wid * RW, ACC // 2)], semo).wait()

    pltpu.sync_copy(idx_hbm.at[wid, pl.ds(0, QI)], idxq.at[0])
    pltpu.async_copy(idx_hbm.at[wid, pl.ds(QI, QI)], idxq.at[1], semi)
    for s in range(3):
        fire(s, s)

    def body2(v, _):
        r0 = 4 * v

        @pl.when(v >= 2)
        def _():
            drain_out()

        @pl.when(jnp.logical_and(v % 32 == 0,
                                 jnp.logical_and(v > 0, v < 96)))
        def _():
            qq = v // 32 + 1
            pltpu.async_copy(
                idx_hbm.at[wid, pl.ds(QI * qq, QI)], idxq.at[qq % 2], semi)

        @pl.when(jnp.logical_and(v % 32 == 31, v < VB - 1))
        def _():
            pltpu.make_async_copy(
                idx_hbm.at[wid, pl.ds(0, QI)], idxq.at[0], semi).wait()

        fire(r0 + 3, 3)
        drain(0)
        finish(r0, 0)

        @pl.when(v < VB - 1)
        def _():
            fire(r0 + 4, 0)

        drain(1)
        finish(r0 + 1, 1)

        @pl.when(v < VB - 1)
        def _():
            fire(r0 + 5, 1)

        drain(2)
        finish(r0 + 2, 2)

        @pl.when(v < VB - 1)
        def _():
            fire(r0 + 6, 2)

        drain(3)
        finish(r0 + 3, 3)

        pltpu.async_copy(
            accv.at[pl.ds((v % 2) * (ACC // 2), ACC // 2)],
            out_hbm.at[pl.ds(wid * RW + r0, ACC // 2)], semo)
        return 0

    lax.fori_loop(0, VB, body2, 0)
    drain_out()
    drain_out()


def _mlp_body(p_ref, w1_ref, b1_ref, w2_ref, b2_ref, w3_ref, b3_ref, o_ref):
    h = jnp.dot(p_ref[...], w1_ref[...], preferred_element_type=jnp.float32)
    h = jnp.maximum(h + b1_ref[...], 0.0)
    h = jnp.dot(h, w2_ref[...], preferred_element_type=jnp.float32)
    h = jnp.maximum(h + b2_ref[...], 0.0)
    o_ref[...] = (
        jnp.dot(h, w3_ref[...], preferred_element_type=jnp.float32)
        + b3_ref[...])


_BT = 1024


def _mlp(pooled, W1, b1, W2, b2, W3, b3):
    full = lambda shape: pl.BlockSpec(shape, lambda i: (0, 0))
    return pl.pallas_call(
        _mlp_body,
        grid=(B // _BT,),
        in_specs=[
            pl.BlockSpec((_BT, EMB), lambda i: (i, 0)),
            full(W1.shape), full((1, 512)),
            full(W2.shape), full((1, 256)),
            full(W3.shape), full((1, OUT)),
        ],
        out_specs=pl.BlockSpec((_BT, OUT), lambda i: (i, 0)),
        out_shape=jax.ShapeDtypeStruct((B, OUT), jnp.float32),
    )(pooled, W1, b1.reshape(1, 512), W2, b2.reshape(1, 256),
      W3, b3.reshape(1, OUT))


def kernel(x, table, W1, b1, W2, b2, W3, b3):
    idx = x.astype(jnp.int32).reshape(NW, 2 * RW, HALF)
    pooled = _sc_pool(idx, table)
    return _mlp(pooled, W1, b1, W2, b2, W3, b3)
